# baseline (speedup 1.0000x reference)
import functools

import jax
import jax.numpy as jnp
from jax import lax
from jax.experimental import pallas as pl
from jax.experimental.pallas import tpu as pltpu
from jax.experimental.pallas import tpu_sc as plsc

D = 128
CHUNK = 64
NW = 32
CPW = 80
NSLOT = 5
E_PAD = NW * CPW * CHUNK
GROUPS = CHUNK // 16


def _sc_body(table, idx0_h, idx1_h, out_h,
             idx0_v, idx1_v, out_v, ra, rb,
             sa0, sb0, sa1, sb1, sa2, sb2, sa3, sb3, sa4, sb4):
    c = lax.axis_index("c")
    s = lax.axis_index("s")
    w = s * 2 + c
    base_chunk = w * CPW

    pltpu.sync_copy(idx0_h.at[pl.ds(base_chunk, CPW)], idx0_v)
    pltpu.sync_copy(idx1_h.at[pl.ds(base_chunk, CPW)], idx1_v)

    sems = ((sa0, sb0), (sa1, sb1), (sa2, sb2), (sa3, sb3), (sa4, sb4))

    def issue(j, slot):
        for q in range(CHUNK // 16):
            iv0 = idx0_v[j, pl.ds(q * 16, 16)]
            iv1 = idx1_v[j, pl.ds(q * 16, 16)]
            pltpu.async_copy(table.at[iv0],
                             ra.at[slot].at[pl.ds(q * 16, 16)],
                             sems[slot][0])
            pltpu.async_copy(table.at[iv1],
                             rb.at[slot].at[pl.ds(q * 16, 16)],
                             sems[slot][1])

    def wait(j, slot):
        pltpu.make_async_copy(table.at[pl.ds(0, CHUNK)], ra.at[slot],
                              sems[slot][0]).wait()
        pltpu.make_async_copy(table.at[pl.ds(0, CHUNK)], rb.at[slot],
                              sems[slot][1]).wait()

    for slot in range(NSLOT):
        issue(slot, slot)

    lanes = lax.broadcasted_iota(jnp.int32, (16,), 0)

    def chunk_grp(jg, carry):
        for slot in range(NSLOT):
            j = jg * NSLOT + slot
            wait(j, slot)
            ra_s = ra.at[slot]
            rb_s = rb.at[slot]

            def group(g, carry2):
                res = jnp.zeros((16,), jnp.float32)
                for e in range(16):
                    row = g * 16 + e
                    acc = None
                    for k in range(4):
                        a2 = plsc.bitcast(ra_s[row, pl.ds(k * 16, 16)],
                                          jnp.bfloat16)
                        b2 = plsc.bitcast(rb_s[row, pl.ds(k * 16, 16)],
                                          jnp.bfloat16)
                        al, ah = plsc.unpack(
                            a2, format=plsc.PackFormat.INTERLEAVED)
                        bl, bh = plsc.unpack(
                            b2, format=plsc.PackFormat.INTERLEAVED)
                        t2 = al * bl + ah * bh
                        acc = t2 if acc is None else acc + t2
                    t = jnp.sum(acc)
                    res = jnp.where(lanes == e, t, res)
                out_v[pl.ds(j * CHUNK + g * 16, 16)] = res
                return carry2

            lax.fori_loop(0, GROUPS, group, 0)

            @pl.when(j + NSLOT < CPW)
            def _():
                issue(j + NSLOT, slot)
        return carry

    lax.fori_loop(0, CPW // NSLOT, chunk_grp, 0)

    pltpu.sync_copy(out_v, out_h.at[pl.ds(base_chunk * CHUNK, CPW * CHUNK)])


_sc_dot = functools.partial(
    pl.kernel,
    out_type=jax.ShapeDtypeStruct((E_PAD,), jnp.float32),
    mesh=plsc.VectorSubcoreMesh(core_axis_name="c", subcore_axis_name="s"),
    compiler_params=pltpu.CompilerParams(needs_layout_passes=False, use_tc_tiling_on_sc=False),
    scratch_types=[
        pltpu.VMEM((CPW, CHUNK), jnp.int32),
        pltpu.VMEM((CPW, CHUNK), jnp.int32),
        pltpu.VMEM((CPW * CHUNK,), jnp.float32),
        pltpu.VMEM((NSLOT, CHUNK, D // 2), jnp.uint32),
        pltpu.VMEM((NSLOT, CHUNK, D // 2), jnp.uint32),
    ] + [pltpu.SemaphoreType.DMA] * (2 * NSLOT),
)(_sc_body)


def kernel(node_feature, edge_label_index, edge_label):
    nf_bf = node_feature.astype(jnp.bfloat16)
    node_feature = lax.bitcast_convert_type(
        nf_bf.reshape(nf_bf.shape[0], D // 2, 2), jnp.uint32)
    e_sup = edge_label_index.shape[1] // 2
    idx = edge_label_index[:, :e_sup]
    pad = E_PAD - e_sup
    idx0 = jnp.pad(idx[0], (0, pad)).reshape(NW * CPW, CHUNK)
    idx1 = jnp.pad(idx[1], (0, pad)).reshape(NW * CPW, CHUNK)
    pred = _sc_dot(node_feature, idx0, idx1)[:e_sup]
    label = edge_label[:e_sup]
    return (pred, label)


# speedup vs baseline: 3.0618x; 3.0618x over previous
import functools

import jax
import jax.numpy as jnp
from jax import lax
from jax.experimental import pallas as pl
from jax.experimental.pallas import tpu as pltpu
from jax.experimental.pallas import tpu_sc as plsc

D = 128
CHUNK = 64
NW = 32
CPW = 80
NSLOT = 5
E_PAD = NW * CPW * CHUNK
GROUPS = CHUNK // 16


def _sc_body(table, eli_h, out_h,
             idx0_v, idx1_v, out_v, ra, rb,
             sa0, sb0, sa1, sb1, sa2, sb2, sa3, sb3, sa4, sb4):
    c = lax.axis_index("c")
    s = lax.axis_index("s")
    w = s * 2 + c
    base_chunk = w * CPW

    base_e = base_chunk * CHUNK
    pltpu.sync_copy(eli_h.at[0].at[pl.ds(base_e, CPW * CHUNK)], idx0_v)
    pltpu.sync_copy(eli_h.at[1].at[pl.ds(base_e, CPW * CHUNK)], idx1_v)

    sems = ((sa0, sb0), (sa1, sb1), (sa2, sb2), (sa3, sb3), (sa4, sb4))

    def issue(j, slot):
        for q in range(CHUNK // 16):
            iv0 = idx0_v[pl.ds(j * CHUNK + q * 16, 16)]
            iv1 = idx1_v[pl.ds(j * CHUNK + q * 16, 16)]
            pltpu.async_copy(table.at[iv0],
                             ra.at[slot].at[pl.ds(q * 16, 16)],
                             sems[slot][0])
            pltpu.async_copy(table.at[iv1],
                             rb.at[slot].at[pl.ds(q * 16, 16)],
                             sems[slot][1])

    def wait(j, slot):
        pltpu.make_async_copy(table.at[pl.ds(0, CHUNK)], ra.at[slot],
                              sems[slot][0]).wait()
        pltpu.make_async_copy(table.at[pl.ds(0, CHUNK)], rb.at[slot],
                              sems[slot][1]).wait()

    for slot in range(NSLOT):
        issue(slot, slot)

    lanes = lax.broadcasted_iota(jnp.int32, (16,), 0)

    def chunk_grp(jg, carry):
        for slot in range(NSLOT):
            j = jg * NSLOT + slot
            wait(j, slot)
            ra_s = ra.at[slot]
            rb_s = rb.at[slot]

            def group(g, carry2):
                res = jnp.zeros((16,), jnp.float32)
                for e in range(16):
                    row = g * 16 + e
                    acc = None
                    for k in range(4):
                        a2 = ra_s[row, pl.ds(k * 32, 32)]
                        b2 = rb_s[row, pl.ds(k * 32, 32)]
                        al, ah = plsc.unpack(
                            a2, format=plsc.PackFormat.INTERLEAVED)
                        bl, bh = plsc.unpack(
                            b2, format=plsc.PackFormat.INTERLEAVED)
                        t2 = al * bl + ah * bh
                        acc = t2 if acc is None else acc + t2
                    t = jnp.sum(acc)
                    res = jnp.where(lanes == e, t, res)
                out_v[pl.ds(j * CHUNK + g * 16, 16)] = res
                return carry2

            lax.fori_loop(0, GROUPS, group, 0)

            @pl.when(j + NSLOT < CPW)
            def _():
                issue(j + NSLOT, slot)
        return carry

    lax.fori_loop(0, CPW // NSLOT, chunk_grp, 0)

    pltpu.sync_copy(out_v, out_h.at[pl.ds(base_chunk * CHUNK, CPW * CHUNK)])


_sc_dot = functools.partial(
    pl.kernel,
    out_type=jax.ShapeDtypeStruct((E_PAD,), jnp.float32),
    mesh=plsc.VectorSubcoreMesh(core_axis_name="c", subcore_axis_name="s"),
    compiler_params=pltpu.CompilerParams(needs_layout_passes=False, use_tc_tiling_on_sc=False),
    scratch_types=[
        pltpu.VMEM((CPW * CHUNK,), jnp.int32),
        pltpu.VMEM((CPW * CHUNK,), jnp.int32),
        pltpu.VMEM((CPW * CHUNK,), jnp.float32),
        pltpu.VMEM((NSLOT, CHUNK, D), jnp.bfloat16),
        pltpu.VMEM((NSLOT, CHUNK, D), jnp.bfloat16),
    ] + [pltpu.SemaphoreType.DMA] * (2 * NSLOT),
)(_sc_body)


def kernel(node_feature, edge_label_index, edge_label):
    node_feature = node_feature.astype(jnp.bfloat16)
    e_sup = edge_label_index.shape[1] // 2
    pred = _sc_dot(node_feature, edge_label_index)[:e_sup]
    label = edge_label[:e_sup]
    return (pred, label)
